# baseline (speedup 1.0000x reference)
import functools

import jax
import jax.numpy as jnp
from jax import lax
from jax.experimental import pallas as pl
from jax.experimental.pallas import tpu as pltpu
from jax.experimental.pallas import tpu_sc as plsc

NC = 2
NS = 16
L = 16
NW = NC * NS

EMBED_DIM = 32
PACK = 128 // EMBED_DIM
BATCH = 16384
BW = BATCH // NW
CHUNK = 128
NCH = BW // CHUNK
HALF = 2
GPC = CHUNK // L


def _nsqrt(x):
    xi = lax.bitcast_convert_type(x, jnp.int32)
    yi = lax.shift_right_logical(xi, 1) + jnp.int32(0x1FBD1DF5)
    y = lax.bitcast_convert_type(yi, jnp.float32)
    for _ in range(3):
        y = 0.5 * (y + x / y)
    return -y


def _body(h_hbm, t_hbm, r_hbm, ent_hbm, rel_hbm, out_hbm,
          rawh0, rawt0, rawr0, rawh1, rawt1, rawr1,
          gh0, gt0, gr0, gh1, gt1, gr1,
          rh0, rt0, rr0, rh1, rt1, rr1,
          out_v, sem):
    wid = lax.axis_index("s") * NC + lax.axis_index("c")
    base = wid * BW
    lane = lax.iota(jnp.int32, L)

    raws = ((rawh0, rawt0, rawr0), (rawh1, rawt1, rawr1))
    gidx = ((gh0, gt0, gr0), (gh1, gt1, gr1))
    rows = ((rh0, rt0, rr0), (rh1, rt1, rr1))
    srcs = (h_hbm, t_hbm, r_hbm)
    tabs = (ent_hbm, ent_hbm, rel_hbm)

    for half in range(NCH // HALF):
        copies = []
        for c2 in range(HALF):
            c = half * HALF + c2
            off = base + c * CHUNK
            for role in range(3):
                pltpu.sync_copy(srcs[role].at[pl.ds(off, CHUNK)], raws[c2][role])

                def shift_grp(g, carry, c2=c2, role=role):
                    v = raws[c2][role][pl.ds(g * L, L)]
                    gidx[c2][role][pl.ds(g * L, L)] = lax.shift_right_logical(v, 2)
                    return carry

                lax.fori_loop(0, GPC, shift_grp, 0)
        for c2 in range(HALF):
            for role in range(3):
                copies.append(pltpu.async_copy(
                    tabs[role].at[gidx[c2][role]], rows[c2][role], sem))
        for cp in copies:
            cp.wait()

        for c2 in range(HALF):
            c = half * HALF + c2

            def group_body(g, carry, c2=c2, c=c):
                rb = g * L
                hv = raws[c2][0][pl.ds(rb, L)]
                tv = raws[c2][1][pl.ds(rb, L)]
                rv = raws[c2][2][pl.ds(rb, L)]
                acc = jnp.zeros((L,), jnp.float32)
                for j in range(L):
                    i = rb + j
                    ho = (hv[j] & (PACK - 1)) * EMBED_DIM
                    to = (tv[j] & (PACK - 1)) * EMBED_DIM
                    ro = (rv[j] & (PACK - 1)) * EMBED_DIM
                    h0 = rows[c2][0][i, pl.ds(ho, L)]
                    h1 = rows[c2][0][i, pl.ds(ho + L, L)]
                    t0 = rows[c2][1][i, pl.ds(to, L)]
                    t1 = rows[c2][1][i, pl.ds(to + L, L)]
                    r0 = rows[c2][2][i, pl.ds(ro, L)]
                    r1 = rows[c2][2][i, pl.ds(ro + L, L)]
                    d0 = (h0 - t0) + r0
                    d1 = (h1 - t1) + r1
                    sq = d0 * d0 + d1 * d1
                    acc = jnp.where(lane == j, jnp.sum(sq), acc)
                out_v[pl.ds(c * CHUNK + rb, L)] = _nsqrt(acc + 1e-12)
                return carry

            lax.fori_loop(0, GPC, group_body, 0)

    pltpu.sync_copy(out_v, out_hbm.at[pl.ds(base, BW)])


_sc_call = functools.partial(
    pl.kernel,
    mesh=plsc.VectorSubcoreMesh(core_axis_name="c", subcore_axis_name="s"),
    out_type=jax.ShapeDtypeStruct((BATCH,), jnp.float32),
    compiler_params=pltpu.CompilerParams(needs_layout_passes=False),
    scratch_types=(
        [pltpu.VMEM((CHUNK,), jnp.int32) for _ in range(6)]
        + [pltpu.VMEM((CHUNK,), jnp.int32) for _ in range(6)]
        + [pltpu.VMEM((CHUNK, 128), jnp.float32) for _ in range(6)]
        + [pltpu.VMEM((BW,), jnp.float32), pltpu.SemaphoreType.DMA]
    ),
)(_body)


@jax.jit
def kernel(triples, entity_table, relation_table):
    h = triples[0]
    t = triples[1]
    r = triples[2]
    ent4 = entity_table.reshape(entity_table.shape[0] // PACK, 128)
    rel4 = relation_table.reshape(relation_table.shape[0] // PACK, 128)
    return _sc_call(h, t, r, ent4, rel4)


# speedup vs baseline: 1.0067x; 1.0067x over previous
import functools

import jax
import jax.numpy as jnp
from jax import lax
from jax.experimental import pallas as pl
from jax.experimental.pallas import tpu as pltpu
from jax.experimental.pallas import tpu_sc as plsc

NC = 2
NS = 16
L = 16
NW = NC * NS

EMBED_DIM = 32
BATCH = 16384
BW = BATCH // NW
CHUNK = 128
NCH = BW // CHUNK
GPC = CHUNK // L


def _nsqrt(x):
    xi = lax.bitcast_convert_type(x, jnp.int32)
    yi = lax.shift_right_logical(xi, 1) + jnp.int32(0x1FBD1DF5)
    y = lax.bitcast_convert_type(yi, jnp.float32)
    for _ in range(3):
        y = 0.5 * (y + x / y)
    return -y


def _body(h_hbm, t_hbm, r_hbm, ent_hbm, rel_hbm, out_hbm,
          h_idx, t_idx, r_idx, rows_h, rows_t, rows_r, out_v, sem):
    wid = lax.axis_index("s") * NC + lax.axis_index("c")
    base = wid * BW

    for c in range(NCH):
        off = base + c * CHUNK
        pltpu.sync_copy(h_hbm.at[pl.ds(off, CHUNK)], h_idx.at[c])
        pltpu.sync_copy(t_hbm.at[pl.ds(off, CHUNK)], t_idx.at[c])
        pltpu.sync_copy(r_hbm.at[pl.ds(off, CHUNK)], r_idx.at[c])

    copies = []
    for c in range(NCH):
        copies.append(pltpu.async_copy(ent_hbm.at[h_idx.at[c]], rows_h.at[c], sem))
        copies.append(pltpu.async_copy(ent_hbm.at[t_idx.at[c]], rows_t.at[c], sem))
        copies.append(pltpu.async_copy(rel_hbm.at[r_idx.at[c]], rows_r.at[c], sem))
    for cp in copies:
        cp.wait()

    lane = lax.iota(jnp.int32, L)

    for c in range(NCH):
        def group_body(g, carry):
            rb = g * L
            acc = jnp.zeros((L,), jnp.float32)
            for j in range(L):
                i = rb + j
                h0 = rows_h[c, i, 0:L]
                h1 = rows_h[c, i, L:EMBED_DIM]
                t0 = rows_t[c, i, 0:L]
                t1 = rows_t[c, i, L:EMBED_DIM]
                r0 = rows_r[c, i, 0:L]
                r1 = rows_r[c, i, L:EMBED_DIM]
                d0 = (h0 - t0) + r0
                d1 = (h1 - t1) + r1
                sq = d0 * d0 + d1 * d1
                acc = jnp.where(lane == j, jnp.sum(sq), acc)
            out_v[pl.ds(c * CHUNK + rb, L)] = _nsqrt(acc + 1e-12)
            return carry

        lax.fori_loop(0, GPC, group_body, 0)

    pltpu.sync_copy(out_v, out_hbm.at[pl.ds(base, BW)])


_sc_call = functools.partial(
    pl.kernel,
    mesh=plsc.VectorSubcoreMesh(core_axis_name="c", subcore_axis_name="s"),
    out_type=jax.ShapeDtypeStruct((BATCH,), jnp.float32),
    compiler_params=pltpu.CompilerParams(
        needs_layout_passes=False, use_tc_tiling_on_sc=False
    ),
    scratch_types=[
        pltpu.VMEM((NCH, CHUNK), jnp.int32),
        pltpu.VMEM((NCH, CHUNK), jnp.int32),
        pltpu.VMEM((NCH, CHUNK), jnp.int32),
        pltpu.VMEM((NCH, CHUNK, EMBED_DIM), jnp.float32),
        pltpu.VMEM((NCH, CHUNK, EMBED_DIM), jnp.float32),
        pltpu.VMEM((NCH, CHUNK, EMBED_DIM), jnp.float32),
        pltpu.VMEM((BW,), jnp.float32),
        pltpu.SemaphoreType.DMA,
    ],
)(_body)


@jax.jit
def kernel(triples, entity_table, relation_table):
    h = triples[0]
    t = triples[1]
    r = triples[2]
    return _sc_call(h, t, r, entity_table, relation_table)
